# baseline (speedup 1.0000x reference)
import functools

import jax
import jax.numpy as jnp
import numpy as np
from jax import lax
from jax.experimental import pallas as pl
from jax.experimental.pallas import tpu as pltpu
from jax.experimental.pallas import tpu_sc as plsc

N_ATOMS = 10000
N_EDGES = 320000
EPSILON = 1e-08
MAX_RADIUS = 5.0
NUM_RBF = 32
BETA = float(((2.0 / NUM_RBF) * (1.0 - np.exp(-MAX_RADIUS))) ** (-2))

TROW = 144
PAYW = 352
AUXW = 16
GW = 128
CW = 128
EB = 512
NB = 1000
NCHUNK = N_EDGES // CW
NWORK = 32
CH_PER_W = -(-NCHUNK // NWORK)

def _f32(shape):
    return jax.ShapeDtypeStruct(shape, jnp.float32)


def _sc_kernel(out_type, scratch_types=()):
    def deco(fn):
        @functools.cache
        def build():
            mesh = plsc.VectorSubcoreMesh(core_axis_name="c",
                                          subcore_axis_name="s")
            return pl.kernel(
                fn, mesh=mesh, out_type=out_type,
                scratch_types=list(scratch_types),
                compiler_params=pltpu.CompilerParams(
                    use_tc_tiling_on_sc=False,
                    skip_device_barrier=True))

        def call(*args):
            return build()(*args)
        return call
    return deco


@_sc_kernel(out_type=(_f32((N_EDGES, TROW)), _f32((N_EDGES, TROW))))
def _sc_gather_hx(t_hbm, ij_hbm, ii_hbm, gj_hbm, gi_hbm):
    def body(ij_v, ii_v, gj_v, gi_v):
        pltpu.sync_copy(t_hbm.at[ij_v.at[0]], gj_v)
        pltpu.sync_copy(t_hbm.at[ii_v.at[0]], gi_v)

    pltpu.emit_pipeline(
        body,
        grid=(N_EDGES // GW,),
        in_specs=[pl.BlockSpec((1, GW), lambda i: (0, i)),
                  pl.BlockSpec((1, GW), lambda i: (0, i))],
        out_specs=[pl.BlockSpec((GW, TROW), lambda i: (i, 0)),
                   pl.BlockSpec((GW, TROW), lambda i: (i, 0))],
        core_axis_name=("c", "s"),
        dimension_semantics=(pltpu.PARALLEL,),
    )(ij_hbm, ii_hbm, gj_hbm, gi_hbm)


@_sc_kernel(out_type=_f32((2, N_ATOMS, AUXW)),
            scratch_types=[pltpu.VMEM((1, CW), jnp.int32),
                           pltpu.VMEM((CW, AUXW), jnp.float32),
                           pltpu.VMEM_SHARED((N_ATOMS, AUXW), jnp.float32)])
def _sc_seg_denom(aux_hbm, ii_hbm, zero_hbm, out_hbm, idx_v, rows_v, table_s):
    c = lax.axis_index("c")
    s = lax.axis_index("s")
    wid = c * 16 + s

    @pl.when(s < 10)
    def _():
        pltpu.sync_copy(zero_hbm.at[pl.ds(s * NB, NB)],
                        table_s.at[pl.ds(s * NB, NB)])
    plsc.subcore_barrier()

    @pl.loop(0, CH_PER_W)
    def _(i):
        chunk = wid + i * NWORK

        @pl.when(chunk < NCHUNK)
        def _():
            base = chunk * CW
            pltpu.sync_copy(ii_hbm.at[0, pl.ds(base, CW)], idx_v.at[0])
            pltpu.sync_copy(aux_hbm.at[pl.ds(base, CW)], rows_v)
            pltpu.sync_copy(rows_v, table_s.at[idx_v.at[0]], add=True)

    plsc.subcore_barrier()

    @pl.when(s < 10)
    def _():
        pltpu.sync_copy(table_s.at[pl.ds(s * NB, NB)],
                        out_hbm.at[c, pl.ds(s * NB, NB)])


@_sc_kernel(out_type=_f32((N_EDGES, AUXW)))
def _sc_gather_denom(den_hbm, ii_hbm, out_hbm):
    def body(ii_v, o_v):
        pltpu.sync_copy(den_hbm.at[ii_v.at[0]], o_v)

    pltpu.emit_pipeline(
        body,
        grid=(N_EDGES // GW,),
        in_specs=[pl.BlockSpec((1, GW), lambda i: (0, i))],
        out_specs=[pl.BlockSpec((GW, AUXW), lambda i: (i, 0))],
        core_axis_name=("c", "s"),
        dimension_semantics=(pltpu.PARALLEL,),
    )(ii_hbm, out_hbm)


QW = 96
CW2 = 256
NC2 = N_EDGES // CW2
NCW2 = 80
IR = CW2 // CW


@_sc_kernel(out_type=tuple(_f32((N_ATOMS, QW)) for _ in range(4)),
            scratch_types=[pltpu.VMEM((2, IR, CW), jnp.int32),
                           pltpu.VMEM((2, CW2, QW), jnp.float32),
                           pltpu.VMEM_SHARED((N_ATOMS, QW), jnp.float32),
                           pltpu.SemaphoreType.DMA,
                           pltpu.SemaphoreType.DMA])
def _sc_seg_payload(p0, p1, p2, p3, ii4_hbm, zero_hbm,
                    o0, o1, o2, o3, idx_v, rows_v, table_s, sem0, sem1):
    c = lax.axis_index("c")
    s = lax.axis_index("s")
    sems = (sem0, sem1)

    def issue(p_hbm, b, k, active):
        chunk = s + 16 * k

        @pl.when(jnp.logical_and(active, chunk < NC2))
        def _():
            pltpu.async_copy(ii4_hbm.at[pl.ds(chunk * IR, IR)],
                             idx_v.at[b], sems[b])
            pltpu.async_copy(p_hbm.at[pl.ds(chunk * CW2, CW2)],
                             rows_v.at[b], sems[b])

    def drain_and_scatter(p_hbm, b, k, active):
        chunk = s + 16 * k

        @pl.when(jnp.logical_and(active, chunk < NC2))
        def _():
            pltpu.make_async_copy(ii4_hbm.at[pl.ds(0, IR)],
                                  idx_v.at[b], sems[b]).wait()
            pltpu.make_async_copy(p_hbm.at[pl.ds(0, CW2)],
                                  rows_v.at[b], sems[b]).wait()
            for j in range(IR):
                pltpu.sync_copy(rows_v.at[b, pl.ds(j * CW, CW)],
                                table_s.at[idx_v.at[b, j]], add=True)

    for q, (p_hbm, out_hbm) in enumerate(((p0, o0), (p1, o1),
                                          (p2, o2), (p3, o3))):
        active = c == (q // 2)

        @pl.when(jnp.logical_and(active, s < 10))
        def _():
            pltpu.sync_copy(zero_hbm.at[pl.ds(s * NB, NB)],
                            table_s.at[pl.ds(s * NB, NB)])
        plsc.subcore_barrier()

        issue(p_hbm, 0, 0, active)
        issue(p_hbm, 1, 1, active)

        @pl.loop(0, NCW2 // 2)
        def _(i):
            drain_and_scatter(p_hbm, 0, 2 * i, active)
            issue(p_hbm, 0, 2 * i + 2, active)
            drain_and_scatter(p_hbm, 1, 2 * i + 1, active)
            issue(p_hbm, 1, 2 * i + 3, active)

        plsc.subcore_barrier()

        @pl.when(jnp.logical_and(active, s < 10))
        def _():
            pltpu.sync_copy(table_s.at[pl.ds(s * NB, NB)],
                            out_hbm.at[pl.ds(s * NB, NB)])
        plsc.subcore_barrier()


def _layer_norm(t):
    m = jnp.mean(t, axis=-1, keepdims=True)
    var = jnp.mean((t - m) ** 2, axis=-1, keepdims=True)
    return (t - m) / jnp.sqrt(var + 1e-05)


def _silu(t):
    return t * (1.0 / (1.0 + jnp.exp(-t)))


def _mm(a, b):
    return jnp.dot(a, b, preferred_element_type=jnp.float32)


def _edge_mlp_kernel(gj, gi, pt, wein, bein, mu, w1a, w1b, w1c, b1,
                     w2, b2, wsem, bsem, e64_o, aux_o):
    hj = gj[:, :128]
    hi = gi[:, :128]
    r = gj[:, 128:131] - gi[:, 128:131]
    d = jnp.sqrt(jnp.sum(r * r, axis=1, keepdims=True) + EPSILON)
    dire = r / (d + EPSILON)
    rbf = jnp.exp(-BETA * (jnp.exp(-d) - mu[:]) ** 2)
    f1 = rbf * (_mm(hj, wein[:128]) + _mm(hi, wein[128:]) + bein[:])
    z1 = (_mm(hj, w1a[:128]) + _mm(hi, w1a[128:]) + _mm(f1, w1b[...])
          + d * w1c[:] + b1[:])
    eh = _layer_norm(_silu(z1))
    e64 = _layer_norm(_mm(eh, w2[...]) + b2[:])
    aw = _mm(e64, wsem[...]) + bsem[:]
    aw = jnp.maximum(aw, 0.0) + jnp.minimum(0.0, 2.0 * (jnp.exp(aw * 0.5) - 1.0))
    selfm = pt[:, 0:1] == pt[:, 1:2]
    w_exp = jnp.where(selfm, 0.0, jnp.exp(aw))
    ones = jnp.ones_like(d)
    pad = jnp.zeros((gj.shape[0], AUXW - 8), jnp.float32)
    e64_o[...] = e64
    aux_o[...] = jnp.concatenate([w_exp, dire, ones, pad], axis=1)


def _sum_partials_kernel(p_ref, o_ref):
    o_ref[...] = p_ref[0] + p_ref[1]


def _edge2_kernel(e64, aux, den, rm, sm, wxm, r2, s2, p0, p1, p2, p3):
    att = aux[:, 0:4] / jnp.maximum(den[:, 0:4], 1e-30)
    h_sem = _mm(e64[...], rm[...]) * _mm(att, sm[...])
    mix = _layer_norm(jnp.tanh(_mm(h_sem, wxm[...])))
    comb = _mm(mix, r2[...]) * _mm(aux[:, 4:7], s2[...])
    p0[...] = h_sem[:, 0:96]
    p1[...] = h_sem[:, 96:192]
    p2[...] = jnp.concatenate([h_sem[:, 192:256], comb[:, 0:32]], axis=1)
    p3[...] = jnp.concatenate(
        [comb[:, 32:96], jnp.zeros((comb.shape[0], 32), jnp.float32)], axis=1)


def _node_kernel(h, x, v, s0, s1, s2, s3, dt, wn1h, wn1s, wn1sp, bn1,
                 wn2, bn2, gm, wpn1, bpn1, wpn2, bpn2, wvm1, bvm1, wvm2, vmx,
                 h_o, x_o, v_o):
    h_sem = jnp.concatenate([s0[...], s1[...], s2[:, 0:64]], axis=1)
    counts = dt[:, 7:8]
    safe = jnp.maximum(counts, 1.0)
    comb_mean = jnp.concatenate([s2[:, 64:96], s3[:, 0:64]], axis=1) / safe
    nsq = _mm(comb_mean * comb_mean, gm[...])
    sh = _layer_norm(_silu(_mm(nsq, wpn1[...]) + bpn1[:]))
    h_sp = _layer_norm(_silu(_mm(sh, wpn2[...]) + bpn2[:]))
    n1 = (_mm(h[...], wn1h[...]) + _mm(h_sem, wn1s[...]) + _mm(h_sp, wn1sp[...]) + bn1[:])
    nh = _layer_norm(_silu(n1))
    h_up = h[...] + _layer_norm(_silu(_mm(nh, wn2[...]) + bn2[:]))
    dv = _mm(comb_mean, vmx[...])
    vmh = _layer_norm(_silu(_mm(h_up, wvm1[...]) + bvm1[:]))
    v_scale = 2.0 / (1.0 + jnp.exp(-_mm(vmh, wvm2[...])))
    v_up = v_scale * v[...] + dv
    h_o[...] = h_up
    x_o[...] = x[...] + v_up
    v_o[...] = v_up


def _const_expanders():
    rm = np.zeros((64, 256), np.float32)
    sm = np.zeros((4, 256), np.float32)
    for b in range(64):
        for k in range(4):
            rm[b, b * 4 + k] = 1.0
            sm[k, b * 4 + k] = 1.0
    r2 = np.zeros((32, 96), np.float32)
    s2 = np.zeros((3, 96), np.float32)
    for cc in range(32):
        for dd in range(3):
            r2[cc, cc * 3 + dd] = 1.0
            s2[dd, cc * 3 + dd] = 1.0
    gm = np.zeros((96, 32), np.float32)
    for cc in range(32):
        for dd in range(3):
            gm[cc * 3 + dd, cc] = 1.0
    return rm, sm, r2, s2, gm


_RM, _SM, _R2, _S2, _GM = _const_expanders()


_DBG_STUB = set()


def kernel(h, x, v, pairlist, params):
    n_atoms, n_edges = h.shape[0], pairlist.shape[1]
    assert n_atoms == N_ATOMS and n_edges == N_EDGES

    ii = pairlist[0].reshape(1, n_edges)
    ij = pairlist[1].reshape(1, n_edges)
    pair_t = jnp.concatenate([pairlist[0][:, None], pairlist[1][:, None]],
                             axis=1)

    table = jnp.concatenate(
        [h, jnp.pad(x, ((0, 0), (0, TROW - 131)))], axis=1)

    if "gather_hx" in _DBG_STUB:
        gj, gi = table[ij[0]], table[ii[0]]
    else:
        gj, gi = _sc_gather_hx(table, ij, ii)

    p = params
    row = lambda a: jnp.asarray(a).reshape(1, -1)
    full = lambda arr: pl.BlockSpec(arr.shape, lambda i: (0,) * arr.ndim)
    w1 = p['W_eo1']
    eb_grid = (n_edges // EB,)
    e_in = [gj, gi, pair_t, p['W_edge_in'], row(p['b_edge_in']),
            row(np.linspace(np.exp(-MAX_RADIUS), 1.0, NUM_RBF)
                .astype(np.float32)),
            w1[:256], w1[256:288], row(w1[288]), row(p['b_eo1']),
            p['W_eo2'], row(p['b_eo2']), p['W_sem'], row(p['b_sem'])]
    e_specs = ([pl.BlockSpec((EB, TROW), lambda i: (i, 0)),
                pl.BlockSpec((EB, TROW), lambda i: (i, 0)),
                pl.BlockSpec((EB, 2), lambda i: (i, 0))]
               + [full(a) for a in e_in[3:]])
    e64, aux = pl.pallas_call(
        _edge_mlp_kernel,
        grid=eb_grid,
        in_specs=e_specs,
        out_specs=[pl.BlockSpec((EB, 64), lambda i: (i, 0)),
                   pl.BlockSpec((EB, AUXW), lambda i: (i, 0))],
        out_shape=(_f32((n_edges, 64)), _f32((n_edges, AUXW))),
    )(*e_in)

    zero16 = jnp.zeros((n_atoms, AUXW), jnp.float32)
    if "seg_denom" in _DBG_STUB:
        full_sum = jax.ops.segment_sum(aux, ii[0], num_segments=n_atoms)
        partials = jnp.stack([full_sum, jnp.zeros_like(full_sum)])
    else:
        partials = _sc_seg_denom(aux, ii, zero16)

    den_table = pl.pallas_call(
        _sum_partials_kernel,
        grid=(1,),
        in_specs=[pl.BlockSpec((2, n_atoms, AUXW), lambda i: (0, 0, 0))],
        out_specs=pl.BlockSpec((n_atoms, AUXW), lambda i: (0, 0)),
        out_shape=_f32((n_atoms, AUXW)),
    )(partials)

    if "gather_denom" in _DBG_STUB:
        den_e = den_table[ii[0]]
    else:
        den_e = _sc_gather_denom(den_table, ii)

    e2_in = [e64, aux, den_e, jnp.asarray(_RM), jnp.asarray(_SM),
             p['W_xm'], jnp.asarray(_R2), jnp.asarray(_S2)]
    e2_specs = ([pl.BlockSpec((EB, 64), lambda i: (i, 0)),
                 pl.BlockSpec((EB, AUXW), lambda i: (i, 0)),
                 pl.BlockSpec((EB, AUXW), lambda i: (i, 0))]
                + [full(a) for a in e2_in[3:]])
    payload = pl.pallas_call(
        _edge2_kernel,
        grid=eb_grid,
        in_specs=e2_specs,
        out_specs=[pl.BlockSpec((EB, QW), lambda i: (i, 0))] * 4,
        out_shape=tuple(_f32((n_edges, QW)) for _ in range(4)),
    )(*e2_in)

    zero96 = jnp.zeros((n_atoms, QW), jnp.float32)
    if "seg_payload" in _DBG_STUB:
        seg = tuple(jax.ops.segment_sum(pq, ii[0], num_segments=n_atoms)
                    for pq in payload)
    else:
        ii4 = pairlist[0].reshape(NC2 * IR, CW)
        seg = _sc_seg_payload(*payload, ii4, zero96)

    wn1 = p['W_n1']
    vmx = jnp.kron(p['W_vmix'], jnp.eye(3, dtype=jnp.float32))
    n_in = [h, x, v, *seg, den_table,
            wn1[:128], wn1[128:384], wn1[384:], row(p['b_n1']),
            p['W_n2'], row(p['b_n2']), jnp.asarray(_GM),
            p['W_pn1'], row(p['b_pn1']), p['W_pn2'], row(p['b_pn2']),
            p['W_vm1'], row(p['b_vm1']), p['W_vm2'], vmx]
    n_specs = ([pl.BlockSpec((NB, 128), lambda i: (i, 0)),
                pl.BlockSpec((NB, 3), lambda i: (i, 0)),
                pl.BlockSpec((NB, 3), lambda i: (i, 0))]
               + [pl.BlockSpec((NB, QW), lambda i: (i, 0))] * 4
               + [pl.BlockSpec((NB, AUXW), lambda i: (i, 0))]
               + [full(a) for a in n_in[8:]])
    h_up, x_up, v_up = pl.pallas_call(
        _node_kernel,
        grid=(n_atoms // NB,),
        in_specs=n_specs,
        out_specs=[pl.BlockSpec((NB, 128), lambda i: (i, 0)),
                   pl.BlockSpec((NB, 3), lambda i: (i, 0)),
                   pl.BlockSpec((NB, 3), lambda i: (i, 0))],
        out_shape=(_f32((n_atoms, 128)), _f32((n_atoms, 3)),
                   _f32((n_atoms, 3))),
    )(*n_in)

    return (h_up, x_up, v_up)


# speedup vs baseline: 1.0034x; 1.0034x over previous
import functools

import jax
import jax.numpy as jnp
import numpy as np
from jax import lax
from jax.experimental import pallas as pl
from jax.experimental.pallas import tpu as pltpu
from jax.experimental.pallas import tpu_sc as plsc

N_ATOMS = 10000
N_EDGES = 320000
EPSILON = 1e-08
MAX_RADIUS = 5.0
NUM_RBF = 32
BETA = float(((2.0 / NUM_RBF) * (1.0 - np.exp(-MAX_RADIUS))) ** (-2))

TROW = 144
PAYW = 352
AUXW = 16
GW = 128
CW = 128
EB = 512
NB = 1000
NCHUNK = N_EDGES // CW
NWORK = 32
CH_PER_W = -(-NCHUNK // NWORK)

def _f32(shape):
    return jax.ShapeDtypeStruct(shape, jnp.float32)


def _sc_kernel(out_type, scratch_types=()):
    def deco(fn):
        @functools.cache
        def build():
            mesh = plsc.VectorSubcoreMesh(core_axis_name="c",
                                          subcore_axis_name="s")
            return pl.kernel(
                fn, mesh=mesh, out_type=out_type,
                scratch_types=list(scratch_types),
                compiler_params=pltpu.CompilerParams(
                    use_tc_tiling_on_sc=False))

        def call(*args):
            return build()(*args)
        return call
    return deco


@_sc_kernel(out_type=(_f32((N_EDGES, TROW)), _f32((N_EDGES, TROW))))
def _sc_gather_hx(t_hbm, ij_hbm, ii_hbm, gj_hbm, gi_hbm):
    def body(ij_v, ii_v, gj_v, gi_v):
        pltpu.sync_copy(t_hbm.at[ij_v.at[0]], gj_v)
        pltpu.sync_copy(t_hbm.at[ii_v.at[0]], gi_v)

    pltpu.emit_pipeline(
        body,
        grid=(N_EDGES // GW,),
        in_specs=[pl.BlockSpec((1, GW), lambda i: (0, i)),
                  pl.BlockSpec((1, GW), lambda i: (0, i))],
        out_specs=[pl.BlockSpec((GW, TROW), lambda i: (i, 0)),
                   pl.BlockSpec((GW, TROW), lambda i: (i, 0))],
        core_axis_name=("c", "s"),
        dimension_semantics=(pltpu.PARALLEL,),
    )(ij_hbm, ii_hbm, gj_hbm, gi_hbm)


DW = 256
DNC = N_EDGES // DW
DIR_ = DW // CW
DNCW = -(-DNC // 16)
DGC = DNC // 2
DGW = -(-DGC // 16)


@_sc_kernel(out_type=(_f32((N_ATOMS, AUXW)), _f32((N_EDGES, AUXW))),
            scratch_types=[pltpu.VMEM((DIR_, CW), jnp.int32),
                           pltpu.VMEM((DW, AUXW), jnp.float32),
                           pltpu.VMEM_SHARED((N_ATOMS, AUXW), jnp.float32)])
def _sc_denom_fused(aux_hbm, ii4_hbm, zero_hbm, table_hbm, dene_hbm,
                    idx_v, rows_v, table_s):
    c = lax.axis_index("c")
    s = lax.axis_index("s")

    @pl.when(s < 10)
    def _():
        pltpu.sync_copy(zero_hbm.at[pl.ds(s * NB, NB)],
                        table_s.at[pl.ds(s * NB, NB)])
    plsc.subcore_barrier()

    @pl.loop(0, DNCW)
    def _(i):
        chunk = s + i * 16

        @pl.when(chunk < DNC)
        def _():
            pltpu.sync_copy(ii4_hbm.at[pl.ds(chunk * DIR_, DIR_)], idx_v)
            pltpu.sync_copy(aux_hbm.at[pl.ds(chunk * DW, DW)], rows_v)
            for j in range(DIR_):
                pltpu.sync_copy(rows_v.at[pl.ds(j * CW, CW)],
                                table_s.at[idx_v.at[j]], add=True)

    plsc.subcore_barrier()

    @pl.when(jnp.logical_and(c == 0, s < 10))
    def _():
        pltpu.sync_copy(table_s.at[pl.ds(s * NB, NB)],
                        table_hbm.at[pl.ds(s * NB, NB)])

    @pl.loop(0, DGW)
    def _(i):
        w = s + i * 16

        @pl.when(w < DGC)
        def _():
            win = c * DGC + w
            pltpu.sync_copy(ii4_hbm.at[pl.ds(win * DIR_, DIR_)], idx_v)
            for j in range(DIR_):
                pltpu.sync_copy(table_s.at[idx_v.at[j]],
                                rows_v.at[pl.ds(j * CW, CW)])
            pltpu.sync_copy(rows_v, dene_hbm.at[pl.ds(win * DW, DW)])


QW = 96
CW2 = 256
NC2 = N_EDGES // CW2
NCW2 = 80
IR = CW2 // CW


@_sc_kernel(out_type=tuple(_f32((N_ATOMS, QW)) for _ in range(4)),
            scratch_types=[pltpu.VMEM((2, IR, CW), jnp.int32),
                           pltpu.VMEM((2, CW2, QW), jnp.float32),
                           pltpu.VMEM_SHARED((N_ATOMS, QW), jnp.float32),
                           pltpu.SemaphoreType.DMA,
                           pltpu.SemaphoreType.DMA])
def _sc_seg_payload(p0, p1, p2, p3, ii4_hbm, zero_hbm,
                    o0, o1, o2, o3, idx_v, rows_v, table_s, sem0, sem1):
    c = lax.axis_index("c")
    s = lax.axis_index("s")
    sems = (sem0, sem1)

    def issue(p_hbm, b, k, active):
        chunk = s + 16 * k

        @pl.when(jnp.logical_and(active, chunk < NC2))
        def _():
            pltpu.async_copy(ii4_hbm.at[pl.ds(chunk * IR, IR)],
                             idx_v.at[b], sems[b])
            pltpu.async_copy(p_hbm.at[pl.ds(chunk * CW2, CW2)],
                             rows_v.at[b], sems[b])

    def drain_and_scatter(p_hbm, b, k, active):
        chunk = s + 16 * k

        @pl.when(jnp.logical_and(active, chunk < NC2))
        def _():
            pltpu.make_async_copy(ii4_hbm.at[pl.ds(0, IR)],
                                  idx_v.at[b], sems[b]).wait()
            pltpu.make_async_copy(p_hbm.at[pl.ds(0, CW2)],
                                  rows_v.at[b], sems[b]).wait()
            for j in range(IR):
                pltpu.sync_copy(rows_v.at[b, pl.ds(j * CW, CW)],
                                table_s.at[idx_v.at[b, j]], add=True)

    for q, (p_hbm, out_hbm) in enumerate(((p0, o0), (p1, o1),
                                          (p2, o2), (p3, o3))):
        active = c == (q // 2)

        @pl.when(jnp.logical_and(active, s < 10))
        def _():
            pltpu.sync_copy(zero_hbm.at[pl.ds(s * NB, NB)],
                            table_s.at[pl.ds(s * NB, NB)])
        plsc.subcore_barrier()

        issue(p_hbm, 0, 0, active)
        issue(p_hbm, 1, 1, active)

        @pl.loop(0, NCW2 // 2)
        def _(i):
            drain_and_scatter(p_hbm, 0, 2 * i, active)
            issue(p_hbm, 0, 2 * i + 2, active)
            drain_and_scatter(p_hbm, 1, 2 * i + 1, active)
            issue(p_hbm, 1, 2 * i + 3, active)

        plsc.subcore_barrier()

        @pl.when(jnp.logical_and(active, s < 10))
        def _():
            pltpu.sync_copy(table_s.at[pl.ds(s * NB, NB)],
                            out_hbm.at[pl.ds(s * NB, NB)])
        plsc.subcore_barrier()


def _layer_norm(t):
    m = jnp.mean(t, axis=-1, keepdims=True)
    var = jnp.mean((t - m) ** 2, axis=-1, keepdims=True)
    return (t - m) / jnp.sqrt(var + 1e-05)


def _silu(t):
    return t * (1.0 / (1.0 + jnp.exp(-t)))


def _mm(a, b):
    return jnp.dot(a, b, preferred_element_type=jnp.float32)


def _edge_mlp_kernel(gj, gi, pt, wein, bein, mu, w1a, w1b, w1c, b1,
                     w2, b2, wsem, bsem, e64_o, aux_o):
    hj = gj[:, :128]
    hi = gi[:, :128]
    r = gj[:, 128:131] - gi[:, 128:131]
    d = jnp.sqrt(jnp.sum(r * r, axis=1, keepdims=True) + EPSILON)
    dire = r / (d + EPSILON)
    rbf = jnp.exp(-BETA * (jnp.exp(-d) - mu[:]) ** 2)
    f1 = rbf * (_mm(hj, wein[:128]) + _mm(hi, wein[128:]) + bein[:])
    z1 = (_mm(hj, w1a[:128]) + _mm(hi, w1a[128:]) + _mm(f1, w1b[...])
          + d * w1c[:] + b1[:])
    eh = _layer_norm(_silu(z1))
    e64 = _layer_norm(_mm(eh, w2[...]) + b2[:])
    aw = _mm(e64, wsem[...]) + bsem[:]
    aw = jnp.maximum(aw, 0.0) + jnp.minimum(0.0, 2.0 * (jnp.exp(aw * 0.5) - 1.0))
    selfm = pt[:, 0:1] == pt[:, 1:2]
    w_exp = jnp.where(selfm, 0.0, jnp.exp(aw))
    ones = jnp.ones_like(d)
    pad = jnp.zeros((gj.shape[0], AUXW - 8), jnp.float32)
    e64_o[...] = e64
    aux_o[...] = jnp.concatenate([w_exp, dire, ones, pad], axis=1)


def _sum_partials_kernel(p_ref, o_ref):
    o_ref[...] = p_ref[0] + p_ref[1]


def _edge2_kernel(e64, aux, den, rm, sm, wxm, r2, s2, p0, p1, p2, p3):
    att = aux[:, 0:4] / jnp.maximum(den[:, 0:4], 1e-30)
    h_sem = _mm(e64[...], rm[...]) * _mm(att, sm[...])
    mix = _layer_norm(jnp.tanh(_mm(h_sem, wxm[...])))
    comb = _mm(mix, r2[...]) * _mm(aux[:, 4:7], s2[...])
    p0[...] = h_sem[:, 0:96]
    p1[...] = h_sem[:, 96:192]
    p2[...] = jnp.concatenate([h_sem[:, 192:256], comb[:, 0:32]], axis=1)
    p3[...] = jnp.concatenate(
        [comb[:, 32:96], jnp.zeros((comb.shape[0], 32), jnp.float32)], axis=1)


def _node_kernel(h, x, v, s0, s1, s2, s3, dt, wn1h, wn1s, wn1sp, bn1,
                 wn2, bn2, gm, wpn1, bpn1, wpn2, bpn2, wvm1, bvm1, wvm2, vmx,
                 h_o, x_o, v_o):
    h_sem = jnp.concatenate([s0[...], s1[...], s2[:, 0:64]], axis=1)
    counts = dt[:, 7:8]
    safe = jnp.maximum(counts, 1.0)
    comb_mean = jnp.concatenate([s2[:, 64:96], s3[:, 0:64]], axis=1) / safe
    nsq = _mm(comb_mean * comb_mean, gm[...])
    sh = _layer_norm(_silu(_mm(nsq, wpn1[...]) + bpn1[:]))
    h_sp = _layer_norm(_silu(_mm(sh, wpn2[...]) + bpn2[:]))
    n1 = (_mm(h[...], wn1h[...]) + _mm(h_sem, wn1s[...]) + _mm(h_sp, wn1sp[...]) + bn1[:])
    nh = _layer_norm(_silu(n1))
    h_up = h[...] + _layer_norm(_silu(_mm(nh, wn2[...]) + bn2[:]))
    dv = _mm(comb_mean, vmx[...])
    vmh = _layer_norm(_silu(_mm(h_up, wvm1[...]) + bvm1[:]))
    v_scale = 2.0 / (1.0 + jnp.exp(-_mm(vmh, wvm2[...])))
    v_up = v_scale * v[...] + dv
    h_o[...] = h_up
    x_o[...] = x[...] + v_up
    v_o[...] = v_up


def _const_expanders():
    rm = np.zeros((64, 256), np.float32)
    sm = np.zeros((4, 256), np.float32)
    for b in range(64):
        for k in range(4):
            rm[b, b * 4 + k] = 1.0
            sm[k, b * 4 + k] = 1.0
    r2 = np.zeros((32, 96), np.float32)
    s2 = np.zeros((3, 96), np.float32)
    for cc in range(32):
        for dd in range(3):
            r2[cc, cc * 3 + dd] = 1.0
            s2[dd, cc * 3 + dd] = 1.0
    gm = np.zeros((96, 32), np.float32)
    for cc in range(32):
        for dd in range(3):
            gm[cc * 3 + dd, cc] = 1.0
    return rm, sm, r2, s2, gm


_RM, _SM, _R2, _S2, _GM = _const_expanders()


_DBG_STUB = set()


def kernel(h, x, v, pairlist, params):
    n_atoms, n_edges = h.shape[0], pairlist.shape[1]
    assert n_atoms == N_ATOMS and n_edges == N_EDGES

    ii = pairlist[0].reshape(1, n_edges)
    ij = pairlist[1].reshape(1, n_edges)
    pair_t = jnp.concatenate([pairlist[0][:, None], pairlist[1][:, None]],
                             axis=1)

    table = jnp.concatenate(
        [h, jnp.pad(x, ((0, 0), (0, TROW - 131)))], axis=1)

    if "gather_hx" in _DBG_STUB:
        gj, gi = table[ij[0]], table[ii[0]]
    else:
        gj, gi = _sc_gather_hx(table, ij, ii)

    p = params
    row = lambda a: jnp.asarray(a).reshape(1, -1)
    full = lambda arr: pl.BlockSpec(arr.shape, lambda i: (0,) * arr.ndim)
    w1 = p['W_eo1']
    eb_grid = (n_edges // EB,)
    e_in = [gj, gi, pair_t, p['W_edge_in'], row(p['b_edge_in']),
            row(np.linspace(np.exp(-MAX_RADIUS), 1.0, NUM_RBF)
                .astype(np.float32)),
            w1[:256], w1[256:288], row(w1[288]), row(p['b_eo1']),
            p['W_eo2'], row(p['b_eo2']), p['W_sem'], row(p['b_sem'])]
    e_specs = ([pl.BlockSpec((EB, TROW), lambda i: (i, 0)),
                pl.BlockSpec((EB, TROW), lambda i: (i, 0)),
                pl.BlockSpec((EB, 2), lambda i: (i, 0))]
               + [full(a) for a in e_in[3:]])
    e64, aux = pl.pallas_call(
        _edge_mlp_kernel,
        grid=eb_grid,
        in_specs=e_specs,
        out_specs=[pl.BlockSpec((EB, 64), lambda i: (i, 0)),
                   pl.BlockSpec((EB, AUXW), lambda i: (i, 0))],
        out_shape=(_f32((n_edges, 64)), _f32((n_edges, AUXW))),
    )(*e_in)

    zero16 = jnp.zeros((n_atoms, AUXW), jnp.float32)
    ii4 = pairlist[0].reshape(N_EDGES // CW, CW)
    if "seg_denom" in _DBG_STUB:
        den_table = jax.ops.segment_sum(aux, ii[0], num_segments=n_atoms)
        den_e = den_table[ii[0]]
    else:
        den_table, den_e = _sc_denom_fused(aux, ii4, zero16)

    e2_in = [e64, aux, den_e, jnp.asarray(_RM), jnp.asarray(_SM),
             p['W_xm'], jnp.asarray(_R2), jnp.asarray(_S2)]
    e2_specs = ([pl.BlockSpec((EB, 64), lambda i: (i, 0)),
                 pl.BlockSpec((EB, AUXW), lambda i: (i, 0)),
                 pl.BlockSpec((EB, AUXW), lambda i: (i, 0))]
                + [full(a) for a in e2_in[3:]])
    payload = pl.pallas_call(
        _edge2_kernel,
        grid=eb_grid,
        in_specs=e2_specs,
        out_specs=[pl.BlockSpec((EB, QW), lambda i: (i, 0))] * 4,
        out_shape=tuple(_f32((n_edges, QW)) for _ in range(4)),
    )(*e2_in)

    zero96 = jnp.zeros((n_atoms, QW), jnp.float32)
    if "seg_payload" in _DBG_STUB:
        seg = tuple(jax.ops.segment_sum(pq, ii[0], num_segments=n_atoms)
                    for pq in payload)
    else:
        seg = _sc_seg_payload(*payload, ii4, zero96)

    wn1 = p['W_n1']
    vmx = jnp.kron(p['W_vmix'], jnp.eye(3, dtype=jnp.float32))
    n_in = [h, x, v, *seg, den_table,
            wn1[:128], wn1[128:384], wn1[384:], row(p['b_n1']),
            p['W_n2'], row(p['b_n2']), jnp.asarray(_GM),
            p['W_pn1'], row(p['b_pn1']), p['W_pn2'], row(p['b_pn2']),
            p['W_vm1'], row(p['b_vm1']), p['W_vm2'], vmx]
    n_specs = ([pl.BlockSpec((NB, 128), lambda i: (i, 0)),
                pl.BlockSpec((NB, 3), lambda i: (i, 0)),
                pl.BlockSpec((NB, 3), lambda i: (i, 0))]
               + [pl.BlockSpec((NB, QW), lambda i: (i, 0))] * 4
               + [pl.BlockSpec((NB, AUXW), lambda i: (i, 0))]
               + [full(a) for a in n_in[8:]])
    h_up, x_up, v_up = pl.pallas_call(
        _node_kernel,
        grid=(n_atoms // NB,),
        in_specs=n_specs,
        out_specs=[pl.BlockSpec((NB, 128), lambda i: (i, 0)),
                   pl.BlockSpec((NB, 3), lambda i: (i, 0)),
                   pl.BlockSpec((NB, 3), lambda i: (i, 0))],
        out_shape=(_f32((n_atoms, 128)), _f32((n_atoms, 3)),
                   _f32((n_atoms, 3))),
    )(*n_in)

    return (h_up, x_up, v_up)


# speedup vs baseline: 1.0039x; 1.0005x over previous
import functools

import jax
import jax.numpy as jnp
import numpy as np
from jax import lax
from jax.experimental import pallas as pl
from jax.experimental.pallas import tpu as pltpu
from jax.experimental.pallas import tpu_sc as plsc

N_ATOMS = 10000
N_EDGES = 320000
EPSILON = 1e-08
MAX_RADIUS = 5.0
NUM_RBF = 32
BETA = float(((2.0 / NUM_RBF) * (1.0 - np.exp(-MAX_RADIUS))) ** (-2))

TROW = 144
PAYW = 352
AUXW = 16
GW = 128
CW = 128
EB = 512
NB = 1000
NWORK = 32

def _f32(shape):
    return jax.ShapeDtypeStruct(shape, jnp.float32)


def _sc_kernel(out_type, scratch_types=()):
    def deco(fn):
        @functools.cache
        def build():
            mesh = plsc.VectorSubcoreMesh(core_axis_name="c",
                                          subcore_axis_name="s")
            return pl.kernel(
                fn, mesh=mesh, out_type=out_type,
                scratch_types=list(scratch_types),
                compiler_params=pltpu.CompilerParams(
                    use_tc_tiling_on_sc=False))

        def call(*args):
            return build()(*args)
        return call
    return deco


@_sc_kernel(out_type=(_f32((N_EDGES, TROW)), _f32((N_EDGES, TROW))))
def _sc_gather_hx(t_hbm, ij_hbm, ii_hbm, gj_hbm, gi_hbm):
    def body(ij_v, ii_v, gj_v, gi_v):
        pltpu.sync_copy(t_hbm.at[ij_v.at[0]], gj_v)
        pltpu.sync_copy(t_hbm.at[ii_v.at[0]], gi_v)

    pltpu.emit_pipeline(
        body,
        grid=(N_EDGES // GW,),
        in_specs=[pl.BlockSpec((1, GW), lambda i: (0, i)),
                  pl.BlockSpec((1, GW), lambda i: (0, i))],
        out_specs=[pl.BlockSpec((GW, TROW), lambda i: (i, 0)),
                   pl.BlockSpec((GW, TROW), lambda i: (i, 0))],
        core_axis_name=("c", "s"),
        dimension_semantics=(pltpu.PARALLEL,),
    )(ij_hbm, ii_hbm, gj_hbm, gi_hbm)


DW = 256
DNC = N_EDGES // DW
DIR_ = DW // CW
DNCW = -(-DNC // 16)
DGC = DNC // 2
DGW = -(-DGC // 16)


@_sc_kernel(out_type=(_f32((N_ATOMS, AUXW)), _f32((N_EDGES, AUXW))),
            scratch_types=[pltpu.VMEM((DIR_, CW), jnp.int32),
                           pltpu.VMEM((DW, AUXW), jnp.float32),
                           pltpu.VMEM_SHARED((N_ATOMS, AUXW), jnp.float32)])
def _sc_denom_fused(aux_hbm, ii4_hbm, zero_hbm, table_hbm, dene_hbm,
                    idx_v, rows_v, table_s):
    c = lax.axis_index("c")
    s = lax.axis_index("s")

    @pl.when(s < 10)
    def _():
        pltpu.sync_copy(zero_hbm.at[pl.ds(s * NB, NB)],
                        table_s.at[pl.ds(s * NB, NB)])
    plsc.subcore_barrier()

    @pl.loop(0, DNCW)
    def _(i):
        chunk = s + i * 16

        @pl.when(chunk < DNC)
        def _():
            pltpu.sync_copy(ii4_hbm.at[pl.ds(chunk * DIR_, DIR_)], idx_v)
            pltpu.sync_copy(aux_hbm.at[pl.ds(chunk * DW, DW)], rows_v)
            for j in range(DIR_):
                pltpu.sync_copy(rows_v.at[pl.ds(j * CW, CW)],
                                table_s.at[idx_v.at[j]], add=True)

    plsc.subcore_barrier()

    @pl.when(jnp.logical_and(c == 0, s < 10))
    def _():
        pltpu.sync_copy(table_s.at[pl.ds(s * NB, NB)],
                        table_hbm.at[pl.ds(s * NB, NB)])

    @pl.loop(0, DGW)
    def _(i):
        w = s + i * 16

        @pl.when(w < DGC)
        def _():
            win = c * DGC + w
            pltpu.sync_copy(ii4_hbm.at[pl.ds(win * DIR_, DIR_)], idx_v)
            for j in range(DIR_):
                pltpu.sync_copy(table_s.at[idx_v.at[j]],
                                rows_v.at[pl.ds(j * CW, CW)])
            pltpu.sync_copy(rows_v, dene_hbm.at[pl.ds(win * DW, DW)])


QW = 96
CW2 = 256
NC2 = N_EDGES // CW2
NCW2 = 80
IR = CW2 // CW


@_sc_kernel(out_type=tuple(_f32((N_ATOMS, QW)) for _ in range(4)),
            scratch_types=[pltpu.VMEM((2, IR, CW), jnp.int32),
                           pltpu.VMEM((2, CW2, QW), jnp.float32),
                           pltpu.VMEM_SHARED((N_ATOMS, QW), jnp.float32),
                           pltpu.SemaphoreType.DMA,
                           pltpu.SemaphoreType.DMA])
def _sc_seg_payload(p0, p1, p2, p3, ii4_hbm, zero_hbm,
                    o0, o1, o2, o3, idx_v, rows_v, table_s, sem0, sem1):
    c = lax.axis_index("c")
    s = lax.axis_index("s")
    sems = (sem0, sem1)

    def issue(p_hbm, b, k, active):
        chunk = s + 16 * k

        @pl.when(jnp.logical_and(active, chunk < NC2))
        def _():
            pltpu.async_copy(ii4_hbm.at[pl.ds(chunk * IR, IR)],
                             idx_v.at[b], sems[b])
            pltpu.async_copy(p_hbm.at[pl.ds(chunk * CW2, CW2)],
                             rows_v.at[b], sems[b])

    def drain_and_scatter(p_hbm, b, k, active):
        chunk = s + 16 * k

        @pl.when(jnp.logical_and(active, chunk < NC2))
        def _():
            pltpu.make_async_copy(ii4_hbm.at[pl.ds(0, IR)],
                                  idx_v.at[b], sems[b]).wait()
            pltpu.make_async_copy(p_hbm.at[pl.ds(0, CW2)],
                                  rows_v.at[b], sems[b]).wait()
            for j in range(IR):
                pltpu.sync_copy(rows_v.at[b, pl.ds(j * CW, CW)],
                                table_s.at[idx_v.at[b, j]], add=True)

    for q, (p_hbm, out_hbm) in enumerate(((p0, o0), (p1, o1),
                                          (p2, o2), (p3, o3))):
        active = c == (q // 2)

        @pl.when(jnp.logical_and(active, s < 10))
        def _():
            pltpu.sync_copy(zero_hbm.at[pl.ds(s * NB, NB)],
                            table_s.at[pl.ds(s * NB, NB)])
        plsc.subcore_barrier()

        issue(p_hbm, 0, 0, active)
        issue(p_hbm, 1, 1, active)

        @pl.loop(0, NCW2 // 2)
        def _(i):
            drain_and_scatter(p_hbm, 0, 2 * i, active)
            issue(p_hbm, 0, 2 * i + 2, active)
            drain_and_scatter(p_hbm, 1, 2 * i + 1, active)
            issue(p_hbm, 1, 2 * i + 3, active)

        plsc.subcore_barrier()

        @pl.when(jnp.logical_and(active, s < 10))
        def _():
            pltpu.sync_copy(table_s.at[pl.ds(s * NB, NB)],
                            out_hbm.at[pl.ds(s * NB, NB)])
        plsc.subcore_barrier()


def _layer_norm(t):
    m = jnp.mean(t, axis=-1, keepdims=True)
    var = jnp.mean((t - m) ** 2, axis=-1, keepdims=True)
    return (t - m) / jnp.sqrt(var + 1e-05)


def _silu(t):
    return t * (1.0 / (1.0 + jnp.exp(-t)))


def _mm(a, b):
    return jnp.dot(a, b, preferred_element_type=jnp.float32)


def _edge_mlp_kernel(gj, gi, pt, wein, bein, mu, w1a, w1b, w1c, b1,
                     w2, b2, wsem, bsem, e64_o, aux_o):
    hj = gj[:, :128]
    hi = gi[:, :128]
    r = gj[:, 128:131] - gi[:, 128:131]
    d = jnp.sqrt(jnp.sum(r * r, axis=1, keepdims=True) + EPSILON)
    dire = r / (d + EPSILON)
    rbf = jnp.exp(-BETA * (jnp.exp(-d) - mu[:]) ** 2)
    f1 = rbf * (_mm(hj, wein[:128]) + _mm(hi, wein[128:]) + bein[:])
    z1 = (_mm(hj, w1a[:128]) + _mm(hi, w1a[128:]) + _mm(f1, w1b[...])
          + d * w1c[:] + b1[:])
    eh = _layer_norm(_silu(z1))
    e64 = _layer_norm(_mm(eh, w2[...]) + b2[:])
    aw = _mm(e64, wsem[...]) + bsem[:]
    aw = jnp.maximum(aw, 0.0) + jnp.minimum(0.0, 2.0 * (jnp.exp(aw * 0.5) - 1.0))
    selfm = pt[:, 0:1] == pt[:, 1:2]
    w_exp = jnp.where(selfm, 0.0, jnp.exp(aw))
    ones = jnp.ones_like(d)
    pad = jnp.zeros((gj.shape[0], AUXW - 8), jnp.float32)
    e64_o[...] = e64
    aux_o[...] = jnp.concatenate([w_exp, dire, ones, pad], axis=1)


def _edge2_kernel(e64, aux, den, rm, sm, wxm, r2, s2, p0, p1, p2, p3):
    att = aux[:, 0:4] / jnp.maximum(den[:, 0:4], 1e-30)
    h_sem = _mm(e64[...], rm[...]) * _mm(att, sm[...])
    mix = _layer_norm(jnp.tanh(_mm(h_sem, wxm[...])))
    comb = _mm(mix, r2[...]) * _mm(aux[:, 4:7], s2[...])
    p0[...] = h_sem[:, 0:96]
    p1[...] = h_sem[:, 96:192]
    p2[...] = jnp.concatenate([h_sem[:, 192:256], comb[:, 0:32]], axis=1)
    p3[...] = jnp.concatenate(
        [comb[:, 32:96], jnp.zeros((comb.shape[0], 32), jnp.float32)], axis=1)


def _node_kernel(h, x, v, s0, s1, s2, s3, dt, wn1h, wn1s, wn1sp, bn1,
                 wn2, bn2, gm, wpn1, bpn1, wpn2, bpn2, wvm1, bvm1, wvm2, vmx,
                 h_o, x_o, v_o):
    h_sem = jnp.concatenate([s0[...], s1[...], s2[:, 0:64]], axis=1)
    counts = dt[:, 7:8]
    safe = jnp.maximum(counts, 1.0)
    comb_mean = jnp.concatenate([s2[:, 64:96], s3[:, 0:64]], axis=1) / safe
    nsq = _mm(comb_mean * comb_mean, gm[...])
    sh = _layer_norm(_silu(_mm(nsq, wpn1[...]) + bpn1[:]))
    h_sp = _layer_norm(_silu(_mm(sh, wpn2[...]) + bpn2[:]))
    n1 = (_mm(h[...], wn1h[...]) + _mm(h_sem, wn1s[...]) + _mm(h_sp, wn1sp[...]) + bn1[:])
    nh = _layer_norm(_silu(n1))
    h_up = h[...] + _layer_norm(_silu(_mm(nh, wn2[...]) + bn2[:]))
    dv = _mm(comb_mean, vmx[...])
    vmh = _layer_norm(_silu(_mm(h_up, wvm1[...]) + bvm1[:]))
    v_scale = 2.0 / (1.0 + jnp.exp(-_mm(vmh, wvm2[...])))
    v_up = v_scale * v[...] + dv
    h_o[...] = h_up
    x_o[...] = x[...] + v_up
    v_o[...] = v_up


def _const_expanders():
    rm = np.zeros((64, 256), np.float32)
    sm = np.zeros((4, 256), np.float32)
    for b in range(64):
        for k in range(4):
            rm[b, b * 4 + k] = 1.0
            sm[k, b * 4 + k] = 1.0
    r2 = np.zeros((32, 96), np.float32)
    s2 = np.zeros((3, 96), np.float32)
    for cc in range(32):
        for dd in range(3):
            r2[cc, cc * 3 + dd] = 1.0
            s2[dd, cc * 3 + dd] = 1.0
    gm = np.zeros((96, 32), np.float32)
    for cc in range(32):
        for dd in range(3):
            gm[cc * 3 + dd, cc] = 1.0
    return rm, sm, r2, s2, gm


_RM, _SM, _R2, _S2, _GM = _const_expanders()


def kernel(h, x, v, pairlist, params):
    n_atoms, n_edges = h.shape[0], pairlist.shape[1]
    assert n_atoms == N_ATOMS and n_edges == N_EDGES

    ii = pairlist[0].reshape(1, n_edges)
    ij = pairlist[1].reshape(1, n_edges)
    pair_t = jnp.concatenate([pairlist[0][:, None], pairlist[1][:, None]],
                             axis=1)

    table = jnp.concatenate(
        [h, jnp.pad(x, ((0, 0), (0, TROW - 131)))], axis=1)

    gj, gi = _sc_gather_hx(table, ij, ii)

    p = params
    row = lambda a: jnp.asarray(a).reshape(1, -1)
    full = lambda arr: pl.BlockSpec(arr.shape, lambda i: (0,) * arr.ndim)
    w1 = p['W_eo1']
    eb_grid = (n_edges // EB,)
    e_in = [gj, gi, pair_t, p['W_edge_in'], row(p['b_edge_in']),
            row(np.linspace(np.exp(-MAX_RADIUS), 1.0, NUM_RBF)
                .astype(np.float32)),
            w1[:256], w1[256:288], row(w1[288]), row(p['b_eo1']),
            p['W_eo2'], row(p['b_eo2']), p['W_sem'], row(p['b_sem'])]
    e_specs = ([pl.BlockSpec((EB, TROW), lambda i: (i, 0)),
                pl.BlockSpec((EB, TROW), lambda i: (i, 0)),
                pl.BlockSpec((EB, 2), lambda i: (i, 0))]
               + [full(a) for a in e_in[3:]])
    e64, aux = pl.pallas_call(
        _edge_mlp_kernel,
        grid=eb_grid,
        in_specs=e_specs,
        out_specs=[pl.BlockSpec((EB, 64), lambda i: (i, 0)),
                   pl.BlockSpec((EB, AUXW), lambda i: (i, 0))],
        out_shape=(_f32((n_edges, 64)), _f32((n_edges, AUXW))),
    )(*e_in)

    zero16 = jnp.zeros((n_atoms, AUXW), jnp.float32)
    ii4 = pairlist[0].reshape(N_EDGES // CW, CW)
    den_table, den_e = _sc_denom_fused(aux, ii4, zero16)

    e2_in = [e64, aux, den_e, jnp.asarray(_RM), jnp.asarray(_SM),
             p['W_xm'], jnp.asarray(_R2), jnp.asarray(_S2)]
    e2_specs = ([pl.BlockSpec((EB, 64), lambda i: (i, 0)),
                 pl.BlockSpec((EB, AUXW), lambda i: (i, 0)),
                 pl.BlockSpec((EB, AUXW), lambda i: (i, 0))]
                + [full(a) for a in e2_in[3:]])
    payload = pl.pallas_call(
        _edge2_kernel,
        grid=eb_grid,
        in_specs=e2_specs,
        out_specs=[pl.BlockSpec((EB, QW), lambda i: (i, 0))] * 4,
        out_shape=tuple(_f32((n_edges, QW)) for _ in range(4)),
    )(*e2_in)

    zero96 = jnp.zeros((n_atoms, QW), jnp.float32)
    seg = _sc_seg_payload(*payload, ii4, zero96)

    wn1 = p['W_n1']
    vmx = jnp.kron(p['W_vmix'], jnp.eye(3, dtype=jnp.float32))
    n_in = [h, x, v, *seg, den_table,
            wn1[:128], wn1[128:384], wn1[384:], row(p['b_n1']),
            p['W_n2'], row(p['b_n2']), jnp.asarray(_GM),
            p['W_pn1'], row(p['b_pn1']), p['W_pn2'], row(p['b_pn2']),
            p['W_vm1'], row(p['b_vm1']), p['W_vm2'], vmx]
    n_specs = ([pl.BlockSpec((NB, 128), lambda i: (i, 0)),
                pl.BlockSpec((NB, 3), lambda i: (i, 0)),
                pl.BlockSpec((NB, 3), lambda i: (i, 0))]
               + [pl.BlockSpec((NB, QW), lambda i: (i, 0))] * 4
               + [pl.BlockSpec((NB, AUXW), lambda i: (i, 0))]
               + [full(a) for a in n_in[8:]])
    h_up, x_up, v_up = pl.pallas_call(
        _node_kernel,
        grid=(n_atoms // NB,),
        in_specs=n_specs,
        out_specs=[pl.BlockSpec((NB, 128), lambda i: (i, 0)),
                   pl.BlockSpec((NB, 3), lambda i: (i, 0)),
                   pl.BlockSpec((NB, 3), lambda i: (i, 0))],
        out_shape=(_f32((n_atoms, 128)), _f32((n_atoms, 3)),
                   _f32((n_atoms, 3))),
    )(*n_in)

    return (h_up, x_up, v_up)
